# baseline (speedup 1.0000x reference)
import jax
import jax.numpy as jnp
from jax.experimental import pallas as pl
from jax.experimental.pallas import tpu as pltpu

N = 100000
M = 50000
C = 64
H = 64
Wd = 2048

_BLK = 2000


def _fuse_body(p_ref, v_ref, r2_ref, wr_ref, br_ref, wp_ref, bp_ref,
               wv_ref, bv_ref, out_ref):
    p = p_ref[...]
    v = v_ref[...]
    r2 = r2_ref[...]
    wr = wr_ref[...]
    wp = wp_ref[...]
    wv = wv_ref[...]
    br = br_ref[...]
    bp = bp_ref[...]
    bv = bv_ref[...]
    cols = []
    for k in range(3):
        acc = (r2 * wr[k][None, :] + p * wp[k][None, :] + v * wv[k][None, :])
        cols.append(jnp.sum(acc, axis=-1) + br[k] + bp[k] + bv[k])
    a0, a1, a2 = cols
    m = jnp.maximum(jnp.maximum(a0, a1), a2)
    e0 = jnp.exp(a0 - m)
    e1 = jnp.exp(a1 - m)
    e2 = jnp.exp(a2 - m)
    inv = 1.0 / (e0 + e1 + e2)
    out_ref[...] = (r2 * (e0 * inv)[:, None] + p * (e1 * inv)[:, None]
                    + v * (e2 * inv)[:, None])


def _fuse_tc(p_F, v2p, r2p, W_r, b_r, W_p, b_p, W_v, b_v):
    grid = (N // _BLK,)
    blk = pl.BlockSpec((_BLK, C), lambda i: (i, 0))
    wspec = pl.BlockSpec((3, C), lambda i: (0, 0))
    bspec = pl.BlockSpec((3,), lambda i: (0,))
    return pl.pallas_call(
        _fuse_body,
        grid=grid,
        in_specs=[blk, blk, blk, wspec, bspec, wspec, bspec, wspec, bspec],
        out_specs=blk,
        out_shape=jax.ShapeDtypeStruct((N, C), jnp.float32),
    )(p_F, v2p, r2p, W_r, b_r, W_p, b_p, W_v, b_v)


def kernel(r, p_F, v_F, W_r, b_r, W_p, b_p, W_v, b_v, p2v, px, py):
    flat = py * Wd + px
    rT = r.reshape(C, H * Wd).T
    v2p = jnp.take(v_F, p2v, axis=0)
    r2p = jnp.take(rT, flat, axis=0)
    fuse = _fuse_tc(p_F, v2p, r2p, W_r, b_r, W_p, b_p, W_v, b_v)
    v_sum = jax.ops.segment_sum(fuse, p2v, num_segments=M)
    v_cnt = jax.ops.segment_sum(jnp.ones((N, 1), jnp.float32), p2v, num_segments=M)
    v_new = v_sum / jnp.maximum(v_cnt, 1.0)
    r_sum = jnp.zeros((H * Wd, C), jnp.float32).at[flat].add(fuse)
    r_cnt = jnp.zeros((H * Wd, 1), jnp.float32).at[flat].add(1.0)
    r_new = (r_sum / jnp.maximum(r_cnt, 1.0)).T.reshape(C, H, Wd)
    return (r_new, fuse, v_new)


# speedup vs baseline: 1.0701x; 1.0701x over previous
import jax
import jax.numpy as jnp
from jax import lax
from jax.experimental import pallas as pl
from jax.experimental.pallas import tpu as pltpu
from jax.experimental.pallas import tpu_sc as plsc

N = 100000
M = 50000
C = 64
H = 64
Wd = 2048
HW = H * Wd

_NC = 2
_NS = 16
_NW = _NC * _NS
_NPW = 3200
_NPAD = _NW * _NPW
_IB = 128
_SB = 640
_NSUP = _NPW // _SB
_NJ = _SB // _IB

_BLK = 2000

_SHARE = 6256
_CB = 320
_SCH = 20
_VCH = 25600
_RCH = 22528
_ACC = _VCH + 8
_GARB = _VCH
_VOUT = _NC * _VCH
_ROUT = _NC * 3 * _RCH


def _gather_body(vF, rT, p2v, flat, v_out, r_out,
                 idx_v, idx_r, buf_v, buf_r, sem_v, sem_r):
    wid = lax.axis_index("s") * _NC + lax.axis_index("c")
    base = wid * _NPW
    pltpu.sync_copy(p2v.at[wid], idx_v)
    pltpu.sync_copy(flat.at[wid], idx_r)
    for s in range(_NSUP):
        cps = []
        for j in range(_NJ):
            row = s * _NJ + j
            dst = pl.ds(j * _IB, _IB)
            cps.append(pltpu.async_copy(vF.at[idx_v.at[row]], buf_v.at[dst], sem_v))
            cps.append(pltpu.async_copy(rT.at[idx_r.at[row]], buf_r.at[dst], sem_r))
        for cp in cps:
            cp.wait()
        out = pl.ds(base + s * _SB, _SB)
        pltpu.sync_copy(buf_v, v_out.at[out])
        pltpu.sync_copy(buf_r, r_out.at[out])


def _sc_gather(v_F, rT, p2v_pad, flat_pad):
    idx_shape = (_NPW // _IB, _IB)
    mesh = plsc.VectorSubcoreMesh(core_axis_name="c", subcore_axis_name="s")
    f = pl.kernel(
        _gather_body,
        out_type=[jax.ShapeDtypeStruct((_NPAD, C), jnp.float32),
                  jax.ShapeDtypeStruct((_NPAD, C), jnp.float32)],
        mesh=mesh,
        scratch_types=[
            pltpu.VMEM(idx_shape, jnp.int32),
            pltpu.VMEM(idx_shape, jnp.int32),
            pltpu.VMEM((_SB, C), jnp.float32),
            pltpu.VMEM((_SB, C), jnp.float32),
            pltpu.SemaphoreType.DMA,
            pltpu.SemaphoreType.DMA,
        ],
        compiler_params=pltpu.CompilerParams(use_tc_tiling_on_sc=False),
    )
    return f(v_F, rT, p2v_pad, flat_pad)


def _fuse_body(p_ref, v_ref, r2_ref, wr_ref, br_ref, wp_ref, bp_ref,
               wv_ref, bv_ref, out_ref):
    p = p_ref[...]
    v = v_ref[...]
    r2 = r2_ref[...]
    wr = wr_ref[...]
    wp = wp_ref[...]
    wv = wv_ref[...]
    br = br_ref[...]
    bp = bp_ref[...]
    bv = bv_ref[...]
    cols = []
    for k in range(3):
        acc = (r2 * wr[k][None, :] + p * wp[k][None, :] + v * wv[k][None, :])
        cols.append(jnp.sum(acc, axis=-1) + br[k] + bp[k] + bv[k])
    a0, a1, a2 = cols
    m = jnp.maximum(jnp.maximum(a0, a1), a2)
    e0 = jnp.exp(a0 - m)
    e1 = jnp.exp(a1 - m)
    e2 = jnp.exp(a2 - m)
    inv = 1.0 / (e0 + e1 + e2)
    out_ref[...] = (r2 * (e0 * inv)[:, None] + p * (e1 * inv)[:, None]
                    + v * (e2 * inv)[:, None])


def _fuse_tc(p_F, v2p, r2p, W_r, b_r, W_p, b_p, W_v, b_v):
    grid = (N // _BLK,)
    blk = pl.BlockSpec((_BLK, C), lambda i: (i, 0))
    wspec = pl.BlockSpec((3, C), lambda i: (0, 0))
    bspec = pl.BlockSpec((3,), lambda i: (0,))
    return pl.pallas_call(
        _fuse_body,
        grid=grid,
        in_specs=[blk, blk, blk, wspec, bspec, wspec, bspec, wspec, bspec],
        out_specs=blk,
        out_shape=jax.ShapeDtypeStruct((N, C), jnp.float32),
    )(p_F, v2p, r2p, W_r, b_r, W_p, b_p, W_v, b_v)


def _scatter_body(fuse, p2v, flat, z2d, z1d,
                  vs_out, vc_out, rs_out, rc_out,
                  fbuf, icb, lidx, ones,
                  acc, cnt, sem, sem_c):
    cid = lax.axis_index("c")
    tid = lax.axis_index("s")
    base_t = tid * _SHARE

    for k in range(8):
        ones[pl.ds(k * 16, 16)] = jnp.full((16,), 1.0, jnp.float32)
    garb = jnp.full((16,), _GARB, jnp.int32)
    for k in range(_CB // 16, (_CB // _IB + 1) * 8):
        lidx[k // 8, pl.ds((k % 8) * 16, 16)] = garb

    iota = lax.broadcasted_iota(jnp.int32, (16,), 0)

    def one_pass(src_hbm, lo, hi, wo_rows, wo_sum, wo_cnt, wo_base):
        zb = tid * 1600
        pltpu.sync_copy(z2d, acc.at[pl.ds(zb, 640)])
        pltpu.sync_copy(z2d, acc.at[pl.ds(zb + 640, 640)])
        pltpu.sync_copy(z2d.at[pl.ds(0, 320)], acc.at[pl.ds(zb + 1280, 320)])
        pltpu.sync_copy(z1d, cnt.at[pl.ds(zb, 1600)])

        @pl.when(tid == 0)
        def _():
            pltpu.sync_copy(z2d.at[pl.ds(0, 8)], acc.at[pl.ds(_VCH, 8)])
            pltpu.sync_copy(z1d.at[pl.ds(0, 8)], cnt.at[pl.ds(_VCH, 8)])

        plsc.subcore_barrier()

        def chunk(s, carry):
            cbase = base_t + s * _CB
            rb = jnp.minimum(cbase, N - _CB)
            pltpu.sync_copy(fuse.at[pl.ds(rb, _CB)], fbuf.at[pl.ds(0, _CB)])
            pltpu.sync_copy(src_hbm.at[pl.ds(rb, _CB)], icb)
            for k in range(_CB // 16):
                g = icb[pl.ds(k * 16, 16)]
                p_id = rb + k * 16 + iota
                ok = ((g >= lo) & (g < hi) & (p_id >= cbase)
                      & (p_id < base_t + _SHARE))
                loc = jnp.where(ok, g - lo, _GARB)
                lidx[k // 8, pl.ds((k % 8) * 16, 16)] = loc
            cps = []
            for j in range(_CB // _IB + 1):
                cps.append(pltpu.async_copy(
                    fbuf.at[pl.ds(j * _IB, _IB)], acc.at[lidx.at[j]],
                    sem, add=True))
                cps.append(pltpu.async_copy(
                    ones, cnt.at[lidx.at[j]], sem_c, add=True))
            for cp in cps:
                cp.wait()
            return carry

        lax.fori_loop(0, _SCH, chunk, 0)
        plsc.subcore_barrier()

        ob = wo_base + tid * wo_rows
        ab = tid * wo_rows
        done = 0
        while done < wo_rows:
            step = min(640, wo_rows - done)
            pltpu.sync_copy(acc.at[pl.ds(ab + done, step)],
                            wo_sum.at[pl.ds(ob + done, step)])
            done += step
        pltpu.sync_copy(cnt.at[pl.ds(ab, wo_rows)],
                        wo_cnt.at[pl.ds(ob, wo_rows)])
        plsc.subcore_barrier()

    vlo = cid * _VCH
    one_pass(p2v, vlo, vlo + _VCH, 1600, vs_out, vc_out, cid * _VCH)
    for p in range(3):
        g = cid * 3 + p
        rlo = g * _RCH
        one_pass(flat, rlo, rlo + _RCH, 1408, rs_out, rc_out, g * _RCH)


def _sc_scatter(fuse, p2v_pad1, flat_pad1, z2d, z1d):
    mesh = plsc.VectorSubcoreMesh(core_axis_name="c", subcore_axis_name="s")
    f = pl.kernel(
        _scatter_body,
        out_type=[jax.ShapeDtypeStruct((_VOUT, C), jnp.float32),
                  jax.ShapeDtypeStruct((_VOUT,), jnp.float32),
                  jax.ShapeDtypeStruct((_ROUT, C), jnp.float32),
                  jax.ShapeDtypeStruct((_ROUT,), jnp.float32)],
        mesh=mesh,
        scratch_types=[
            pltpu.VMEM((_IB * (_CB // _IB + 1), C), jnp.float32),
            pltpu.VMEM((_CB,), jnp.int32),
            pltpu.VMEM((_CB // _IB + 1, _IB), jnp.int32),
            pltpu.VMEM((_IB,), jnp.float32),
            pltpu.VMEM_SHARED((_ACC, C), jnp.float32),
            pltpu.VMEM_SHARED((_ACC,), jnp.float32),
            pltpu.SemaphoreType.DMA,
            pltpu.SemaphoreType.DMA,
        ],
        compiler_params=pltpu.CompilerParams(use_tc_tiling_on_sc=False),
    )
    return f(fuse, p2v_pad1, flat_pad1, z2d, z1d)


def _post_v_body(vs_ref, vc_ref, out_ref):
    c = jnp.maximum(vc_ref[...], 1.0)
    out_ref[...] = vs_ref[...] / c[:, None]


def _post_v(vs, vc):
    grid = ((M + 1023) // 1024,)
    return pl.pallas_call(
        _post_v_body,
        grid=grid,
        in_specs=[pl.BlockSpec((1024, C), lambda i: (i, 0)),
                  pl.BlockSpec((1024,), lambda i: (i,))],
        out_specs=pl.BlockSpec((1024, C), lambda i: (i, 0)),
        out_shape=jax.ShapeDtypeStruct((M, C), jnp.float32),
    )(vs, vc)


def _post_r_body(rs_ref, rc_ref, out_ref):
    c = jnp.maximum(rc_ref[...], 1.0)
    out_ref[...] = (rs_ref[...] / c[:, None]).T


def _post_r(rs, rc):
    grid = (HW // 1024,)
    return pl.pallas_call(
        _post_r_body,
        grid=grid,
        in_specs=[pl.BlockSpec((1024, C), lambda i: (i, 0)),
                  pl.BlockSpec((1024,), lambda i: (i,))],
        out_specs=pl.BlockSpec((C, 1024), lambda i: (0, i)),
        out_shape=jax.ShapeDtypeStruct((C, HW), jnp.float32),
    )(rs, rc)


def kernel(r, p_F, v_F, W_r, b_r, W_p, b_p, W_v, b_v, p2v, px, py):
    flat = py * Wd + px
    rT = r.reshape(C, HW).T
    p2v_pad = jnp.pad(p2v, (0, _NPAD - N)).reshape(_NW, _NPW // _IB, _IB)
    flat_pad = jnp.pad(flat, (0, _NPAD - N)).reshape(_NW, _NPW // _IB, _IB)
    z2d = jnp.zeros((_SB, C), jnp.float32)
    z1d = jnp.zeros((1600,), jnp.float32)

    v2p_pad, r2p_pad = _sc_gather(v_F, rT, p2v_pad, flat_pad)
    fuse = _fuse_tc(p_F, v2p_pad, r2p_pad, W_r, b_r, W_p, b_p, W_v, b_v)
    vs, vc, rs, rc = _sc_scatter(fuse, p2v, flat, z2d, z1d)
    v_new = _post_v(vs, vc)
    r_new = _post_r(rs, rc).reshape(C, H, Wd)
    return (r_new, fuse, v_new)


# speedup vs baseline: 1.6196x; 1.5135x over previous
import jax
import jax.numpy as jnp
from jax import lax
from jax.experimental import pallas as pl
from jax.experimental.pallas import tpu as pltpu
from jax.experimental.pallas import tpu_sc as plsc

N = 100000
M = 50000
C = 64
H = 64
Wd = 2048
HW = H * Wd

_NC = 2
_NS = 16
_NW = _NC * _NS
_NPW = 3200
_NPAD = _NW * _NPW
_IB = 128
_SB = 640
_NSUP = _NPW // _SB
_NJ = _SB // _IB

_BLK = 2000

_SHARE = 6256
_CH = 22528
_ACC = _CH + 8
_GARB = _CH
_VOUT = 50176
_ROUT = 6 * _CH
_PLCAP = 6272


def _gather_body(vF, rT, p2v, flat, v_out, r_out,
                 idx_v, idx_r, buf_v, buf_r, sem_v, sem_r):
    wid = lax.axis_index("s") * _NC + lax.axis_index("c")
    base = wid * _NPW
    pltpu.sync_copy(p2v.at[wid], idx_v)
    pltpu.sync_copy(flat.at[wid], idx_r)
    for s in range(_NSUP):
        cps = []
        for j in range(_NJ):
            row = s * _NJ + j
            dst = pl.ds(j * _IB, _IB)
            cps.append(pltpu.async_copy(vF.at[idx_v.at[row]], buf_v.at[dst], sem_v))
            cps.append(pltpu.async_copy(rT.at[idx_r.at[row]], buf_r.at[dst], sem_r))
        for cp in cps:
            cp.wait()
        out = pl.ds(base + s * _SB, _SB)
        pltpu.sync_copy(buf_v, v_out.at[out])
        pltpu.sync_copy(buf_r, r_out.at[out])


def _sc_gather(v_F, rT, p2v_pad, flat_pad):
    idx_shape = (_NPW // _IB, _IB)
    mesh = plsc.VectorSubcoreMesh(core_axis_name="c", subcore_axis_name="s")
    f = pl.kernel(
        _gather_body,
        out_type=[jax.ShapeDtypeStruct((_NPAD, C), jnp.float32),
                  jax.ShapeDtypeStruct((_NPAD, C), jnp.float32)],
        mesh=mesh,
        scratch_types=[
            pltpu.VMEM(idx_shape, jnp.int32),
            pltpu.VMEM(idx_shape, jnp.int32),
            pltpu.VMEM((_SB, C), jnp.float32),
            pltpu.VMEM((_SB, C), jnp.float32),
            pltpu.SemaphoreType.DMA,
            pltpu.SemaphoreType.DMA,
        ],
        compiler_params=pltpu.CompilerParams(use_tc_tiling_on_sc=False),
    )
    return f(v_F, rT, p2v_pad, flat_pad)


def _fuse_body(p_ref, v_ref, r2_ref, wr_ref, br_ref, wp_ref, bp_ref,
               wv_ref, bv_ref, out_ref):
    p = p_ref[...]
    v = v_ref[...]
    r2 = r2_ref[...]
    wr = wr_ref[...]
    wp = wp_ref[...]
    wv = wv_ref[...]
    br = br_ref[...]
    bp = bp_ref[...]
    bv = bv_ref[...]
    cols = []
    for k in range(3):
        acc = (r2 * wr[k][None, :] + p * wp[k][None, :] + v * wv[k][None, :])
        cols.append(jnp.sum(acc, axis=-1) + br[k] + bp[k] + bv[k])
    a0, a1, a2 = cols
    m = jnp.maximum(jnp.maximum(a0, a1), a2)
    e0 = jnp.exp(a0 - m)
    e1 = jnp.exp(a1 - m)
    e2 = jnp.exp(a2 - m)
    inv = 1.0 / (e0 + e1 + e2)
    out_ref[...] = (r2 * (e0 * inv)[:, None] + p * (e1 * inv)[:, None]
                    + v * (e2 * inv)[:, None])


def _fuse_tc(p_F, v2p, r2p, W_r, b_r, W_p, b_p, W_v, b_v):
    grid = (N // _BLK,)
    blk = pl.BlockSpec((_BLK, C), lambda i: (i, 0))
    wspec = pl.BlockSpec((3, C), lambda i: (0, 0))
    bspec = pl.BlockSpec((3,), lambda i: (0,))
    return pl.pallas_call(
        _fuse_body,
        grid=grid,
        in_specs=[blk, blk, blk, wspec, bspec, wspec, bspec, wspec, bspec],
        out_specs=blk,
        out_shape=jax.ShapeDtypeStruct((N, C), jnp.float32),
    )(p_F, v2p, r2p, W_r, b_r, W_p, b_p, W_v, b_v)


def _scatter_body(fuse, p2v, flat, z2d, z1d, zi32,
                  vs_out, vc_out, rs_out, rc_out,
                  fbuf, ibuf, pidl, locl, lidx, ones,
                  acc, cnt, sem_g, sem_s, sem_c):
    cid = lax.axis_index("c")
    tid = lax.axis_index("s")
    base_t = tid * _SHARE

    pltpu.sync_copy(zi32, pidl)
    for k in range(8):
        ones[pl.ds(k * 16, 16)] = jnp.full((16,), 1.0, jnp.float32)
    iota = lax.broadcasted_iota(jnp.int32, (16,), 0)

    def one_pass(src_hbm, lo, hi, rows, wo_sum, wo_cnt, wo_base):
        zrows = rows // _NS
        zb = tid * zrows
        done = 0
        while done < zrows:
            step = min(640, zrows - done)
            pltpu.sync_copy(z2d.at[pl.ds(0, step)],
                            acc.at[pl.ds(zb + done, step)])
            done += step
        pltpu.sync_copy(z1d.at[pl.ds(0, zrows)], cnt.at[pl.ds(zb, zrows)])

        @pl.when(tid == 0)
        def _():
            pltpu.sync_copy(z2d.at[pl.ds(0, 8)], acc.at[pl.ds(_CH, 8)])
            pltpu.sync_copy(z1d.at[pl.ds(0, 8)], cnt.at[pl.ds(_CH, 8)])

        pltpu.sync_copy(src_hbm.at[pl.ds(base_t, _PLCAP)], ibuf)

        def binstep(k, off):
            g = ibuf[pl.ds(k * 16, 16)]
            inb = (g >= lo) & (g < hi) & ((k * 16 + iota) < _SHARE)
            ii = inb.astype(jnp.int32)
            csum = plsc.cumsum(ii)
            tgt = jnp.where(inb, off + csum - ii, 0)
            plsc.store_scatter(pidl, [tgt], base_t + k * 16 + iota, mask=inb)
            plsc.store_scatter(locl, [tgt], g - lo, mask=inb)
            return off + plsc.all_reduce_population_count(inb)

        npts_v = lax.fori_loop(0, _PLCAP // 16, binstep,
                               jnp.zeros((16,), jnp.int32))
        npts = npts_v[0]
        plsc.subcore_barrier()

        nb = (npts + _IB - 1) // _IB

        def batch(b, carry):
            bb = b * _IB
            pltpu.async_copy(fuse.at[pidl.at[pl.ds(bb, _IB)]],
                             fbuf, sem_g).wait()
            for j in range(_IB // 16):
                lv = locl[pl.ds(bb + j * 16, 16)]
                pos = bb + j * 16 + iota
                lidx[0, pl.ds(j * 16, 16)] = jnp.where(pos < npts, lv, _GARB)
            c1 = pltpu.async_copy(fbuf, acc.at[lidx.at[0]], sem_s, add=True)
            c2 = pltpu.async_copy(ones, cnt.at[lidx.at[0]], sem_c, add=True)
            c1.wait()
            c2.wait()
            return carry

        lax.fori_loop(0, nb, batch, 0)
        plsc.subcore_barrier()

        wrows = rows // _NS
        ob = wo_base + tid * wrows
        ab = tid * wrows
        done = 0
        while done < wrows:
            step = min(640, wrows - done)
            pltpu.sync_copy(acc.at[pl.ds(ab + done, step)],
                            wo_sum.at[pl.ds(ob + done, step)])
            done += step
        pltpu.sync_copy(cnt.at[pl.ds(ab, wrows)],
                        wo_cnt.at[pl.ds(ob, wrows)])
        plsc.subcore_barrier()

    for p in range(6):
        if p < 3:
            vrows = _CH if p < 2 else _VOUT - 2 * _CH

            @pl.when(cid == 0)
            def _(p=p, vrows=vrows):
                one_pass(p2v, p * _CH, p * _CH + vrows, vrows,
                         vs_out, vc_out, p * _CH)

        @pl.when(cid == 1)
        def _(p=p):
            one_pass(flat, p * _CH, (p + 1) * _CH, _CH,
                     rs_out, rc_out, p * _CH)


def _sc_scatter(fuse, p2v_s, flat_s, z2d, z1d, zi32):
    mesh = plsc.VectorSubcoreMesh(core_axis_name="c", subcore_axis_name="s")
    f = pl.kernel(
        _scatter_body,
        out_type=[jax.ShapeDtypeStruct((_VOUT, C), jnp.float32),
                  jax.ShapeDtypeStruct((_VOUT,), jnp.float32),
                  jax.ShapeDtypeStruct((_ROUT, C), jnp.float32),
                  jax.ShapeDtypeStruct((_ROUT,), jnp.float32)],
        mesh=mesh,
        scratch_types=[
            pltpu.VMEM((_IB, C), jnp.float32),
            pltpu.VMEM((_PLCAP,), jnp.int32),
            pltpu.VMEM((_PLCAP,), jnp.int32),
            pltpu.VMEM((_PLCAP,), jnp.int32),
            pltpu.VMEM((1, _IB), jnp.int32),
            pltpu.VMEM((_IB,), jnp.float32),
            pltpu.VMEM_SHARED((_ACC, C), jnp.float32),
            pltpu.VMEM_SHARED((_ACC,), jnp.float32),
            pltpu.SemaphoreType.DMA,
            pltpu.SemaphoreType.DMA,
            pltpu.SemaphoreType.DMA,
        ],
        compiler_params=pltpu.CompilerParams(use_tc_tiling_on_sc=False,
                                             needs_layout_passes=False),
    )
    return f(fuse, p2v_s, flat_s, z2d, z1d, zi32)


def _post_v_body(vs_ref, vc_ref, out_ref):
    c = jnp.maximum(vc_ref[...], 1.0)
    out_ref[...] = vs_ref[...] / c[:, None]


def _post_v(vs, vc):
    grid = ((M + 1023) // 1024,)
    return pl.pallas_call(
        _post_v_body,
        grid=grid,
        in_specs=[pl.BlockSpec((1024, C), lambda i: (i, 0)),
                  pl.BlockSpec((1024,), lambda i: (i,))],
        out_specs=pl.BlockSpec((1024, C), lambda i: (i, 0)),
        out_shape=jax.ShapeDtypeStruct((M, C), jnp.float32),
    )(vs, vc)


def _post_r_body(rs_ref, rc_ref, out_ref):
    c = jnp.maximum(rc_ref[...], 1.0)
    out_ref[...] = (rs_ref[...] / c[:, None]).T


def _post_r(rs, rc):
    grid = (HW // 1024,)
    return pl.pallas_call(
        _post_r_body,
        grid=grid,
        in_specs=[pl.BlockSpec((1024, C), lambda i: (i, 0)),
                  pl.BlockSpec((1024,), lambda i: (i,))],
        out_specs=pl.BlockSpec((C, 1024), lambda i: (0, i)),
        out_shape=jax.ShapeDtypeStruct((C, HW), jnp.float32),
    )(rs, rc)


def kernel(r, p_F, v_F, W_r, b_r, W_p, b_p, W_v, b_v, p2v, px, py):
    flat = py * Wd + px
    rT = r.reshape(C, HW).T
    p2v_pad = jnp.pad(p2v, (0, _NPAD - N)).reshape(_NW, _NPW // _IB, _IB)
    flat_pad = jnp.pad(flat, (0, _NPAD - N)).reshape(_NW, _NPW // _IB, _IB)
    z2d = jnp.zeros((_SB, C), jnp.float32)
    z1d = jnp.zeros((1600,), jnp.float32)
    zi32 = jnp.zeros((_PLCAP,), jnp.int32)
    sentinel = jnp.int32(1 << 20)
    p2v_s = jnp.pad(p2v, (0, _NPAD - N), constant_values=sentinel)
    flat_s = jnp.pad(flat, (0, _NPAD - N), constant_values=sentinel)

    v2p_pad, r2p_pad = _sc_gather(v_F, rT, p2v_pad, flat_pad)
    fuse = _fuse_tc(p_F, v2p_pad, r2p_pad, W_r, b_r, W_p, b_p, W_v, b_v)
    vs, vc, rs, rc = _sc_scatter(fuse, p2v_s, flat_s, z2d, z1d, zi32)
    v_new = _post_v(vs, vc)
    r_new = _post_r(rs, rc).reshape(C, H, Wd)
    return (r_new, fuse, v_new)


# speedup vs baseline: 1.6744x; 1.0339x over previous
import jax
import jax.numpy as jnp
from jax import lax
from jax.experimental import pallas as pl
from jax.experimental.pallas import tpu as pltpu
from jax.experimental.pallas import tpu_sc as plsc

N = 100000
M = 50000
C = 64
H = 64
Wd = 2048
HW = H * Wd

_NC = 2
_NS = 16
_NW = _NC * _NS
_NPW = 3200
_NPAD = _NW * _NPW
_IB = 128
_SB = 640
_GB = 320
_GSUP = _NPW // _GB
_GJ = _GB // _IB

_BLK = 2000

_SHARE = 6256
_CH = 22528
_ACC = _CH + 8
_GARB = _CH
_VOUT = 50176
_ROUT = 6 * _CH
_PLCAP = 6272


def _gather_body(vF, rT, p2v, flat, v_out, r_out,
                 idx_v, idx_r, buf_v, buf_r, sem_v0, sem_v1, sem_r0, sem_r1,
                 sem_o):
    wid = lax.axis_index("s") * _NC + lax.axis_index("c")
    base = wid * _NPW
    pltpu.sync_copy(p2v.at[wid], idx_v)
    pltpu.sync_copy(flat.at[wid], idx_r)

    def fire(s, par):
        bo = par * _GB
        sv = sem_v1 if par else sem_v0
        sr = sem_r1 if par else sem_r0
        for j in range(_GB // _IB):
            row = (s * _GB) // _IB + j
            dst = pl.ds(bo + j * _IB, _IB)
            pltpu.async_copy(vF.at[idx_v.at[row]], buf_v.at[dst], sv)
            pltpu.async_copy(rT.at[idx_r.at[row]], buf_r.at[dst], sr)

    def drain(s, par):
        bo = par * _GB
        sv = sem_v1 if par else sem_v0
        sr = sem_r1 if par else sem_r0
        for j in range(_GB // _IB):
            dst = pl.ds(bo + j * _IB, _IB)
            pltpu.make_async_copy(vF.at[idx_v.at[0]], buf_v.at[dst], sv).wait()
            pltpu.make_async_copy(rT.at[idx_r.at[0]], buf_r.at[dst], sr).wait()
        out = pl.ds(base + s * _GB, _GB)
        ov = pltpu.async_copy(buf_v.at[pl.ds(bo, _GB)], v_out.at[out], sem_o)
        orr = pltpu.async_copy(buf_r.at[pl.ds(bo, _GB)], r_out.at[out], sem_o)
        return ov, orr

    fire(0, 0)
    pend = []
    for s in range(_GSUP):
        par = s % 2
        for cp in pend:
            cp.wait()
        pend = []
        if s + 1 < _GSUP:
            fire(s + 1, 1 - par)
        pend = list(drain(s, par))
    for cp in pend:
        cp.wait()


def _sc_gather(v_F, rT, p2v_pad, flat_pad):
    idx_shape = (_NPW // _IB, _IB)
    mesh = plsc.VectorSubcoreMesh(core_axis_name="c", subcore_axis_name="s")
    f = pl.kernel(
        _gather_body,
        out_type=[jax.ShapeDtypeStruct((_NPAD, C), jnp.float32),
                  jax.ShapeDtypeStruct((_NPAD, C), jnp.float32)],
        mesh=mesh,
        scratch_types=[
            pltpu.VMEM(idx_shape, jnp.int32),
            pltpu.VMEM(idx_shape, jnp.int32),
            pltpu.VMEM((2 * _GB, C), jnp.float32),
            pltpu.VMEM((2 * _GB, C), jnp.float32),
            pltpu.SemaphoreType.DMA,
            pltpu.SemaphoreType.DMA,
            pltpu.SemaphoreType.DMA,
            pltpu.SemaphoreType.DMA,
            pltpu.SemaphoreType.DMA,
        ],
        compiler_params=pltpu.CompilerParams(use_tc_tiling_on_sc=False),
    )
    return f(v_F, rT, p2v_pad, flat_pad)


def _tr_body(r_ref, out_ref):
    out_ref[...] = r_ref[...].T


def _tr_tc(r2):
    grid = (HW // 1024,)
    return pl.pallas_call(
        _tr_body,
        grid=grid,
        in_specs=[pl.BlockSpec((C, 1024), lambda i: (0, i))],
        out_specs=pl.BlockSpec((1024, C), lambda i: (i, 0)),
        out_shape=jax.ShapeDtypeStruct((HW, C), jnp.float32),
    )(r2)


def _fuse_body(p_ref, v_ref, r2_ref, wr_ref, br_ref, wp_ref, bp_ref,
               wv_ref, bv_ref, out_ref):
    p = p_ref[...]
    v = v_ref[...]
    r2 = r2_ref[...]
    wr = wr_ref[...]
    wp = wp_ref[...]
    wv = wv_ref[...]
    br = br_ref[...]
    bp = bp_ref[...]
    bv = bv_ref[...]
    cols = []
    for k in range(3):
        acc = (r2 * wr[k][None, :] + p * wp[k][None, :] + v * wv[k][None, :])
        cols.append(jnp.sum(acc, axis=-1) + br[k] + bp[k] + bv[k])
    a0, a1, a2 = cols
    m = jnp.maximum(jnp.maximum(a0, a1), a2)
    e0 = jnp.exp(a0 - m)
    e1 = jnp.exp(a1 - m)
    e2 = jnp.exp(a2 - m)
    inv = 1.0 / (e0 + e1 + e2)
    out_ref[...] = (r2 * (e0 * inv)[:, None] + p * (e1 * inv)[:, None]
                    + v * (e2 * inv)[:, None])


def _fuse_tc(p_F, v2p, r2p, W_r, b_r, W_p, b_p, W_v, b_v):
    grid = (N // _BLK,)
    blk = pl.BlockSpec((_BLK, C), lambda i: (i, 0))
    wspec = pl.BlockSpec((3, C), lambda i: (0, 0))
    bspec = pl.BlockSpec((3,), lambda i: (0,))
    return pl.pallas_call(
        _fuse_body,
        grid=grid,
        in_specs=[blk, blk, blk, wspec, bspec, wspec, bspec, wspec, bspec],
        out_specs=blk,
        out_shape=jax.ShapeDtypeStruct((N, C), jnp.float32),
    )(p_F, v2p, r2p, W_r, b_r, W_p, b_p, W_v, b_v)


def _scatter_body(fuse, p2v, flat, z2d, z1d, zi32,
                  vs_out, vc_out, rs_out, rc_out,
                  fbuf, ibuf, pidl, locl, lidx, ones,
                  acc, cnt, sem_g, sem_s, sem_c):
    cid = lax.axis_index("c")
    tid = lax.axis_index("s")
    base_t = tid * _SHARE

    pltpu.sync_copy(zi32, pidl)
    for k in range(8):
        ones[pl.ds(k * 16, 16)] = jnp.full((16,), 1.0, jnp.float32)
    iota = lax.broadcasted_iota(jnp.int32, (16,), 0)

    def one_pass(src_hbm, lo, hi, rows, wo_sum, wo_cnt, wo_base):
        zrows = rows // _NS
        zb = tid * zrows
        done = 0
        while done < zrows:
            step = min(640, zrows - done)
            pltpu.sync_copy(z2d.at[pl.ds(0, step)],
                            acc.at[pl.ds(zb + done, step)])
            done += step
        pltpu.sync_copy(z1d.at[pl.ds(0, zrows)], cnt.at[pl.ds(zb, zrows)])

        @pl.when(tid == 0)
        def _():
            pltpu.sync_copy(z2d.at[pl.ds(0, 8)], acc.at[pl.ds(_CH, 8)])
            pltpu.sync_copy(z1d.at[pl.ds(0, 8)], cnt.at[pl.ds(_CH, 8)])

        pltpu.sync_copy(src_hbm.at[pl.ds(base_t, _PLCAP)], ibuf)

        def binstep(k, off):
            g = ibuf[pl.ds(k * 16, 16)]
            inb = (g >= lo) & (g < hi) & ((k * 16 + iota) < _SHARE)
            ii = inb.astype(jnp.int32)
            csum = plsc.cumsum(ii)
            tgt = jnp.where(inb, off + csum - ii, 0)
            plsc.store_scatter(pidl, [tgt], base_t + k * 16 + iota, mask=inb)
            plsc.store_scatter(locl, [tgt], g - lo, mask=inb)
            return off + plsc.all_reduce_population_count(inb)

        npts_v = lax.fori_loop(0, _PLCAP // 16, binstep,
                               jnp.zeros((16,), jnp.int32))
        npts = npts_v[0]
        plsc.subcore_barrier()

        nb = (npts + _IB - 1) // _IB

        def batch(b, carry):
            bb = b * _IB
            pltpu.async_copy(fuse.at[pidl.at[pl.ds(bb, _IB)]],
                             fbuf, sem_g).wait()
            for j in range(_IB // 16):
                lv = locl[pl.ds(bb + j * 16, 16)]
                pos = bb + j * 16 + iota
                lidx[0, pl.ds(j * 16, 16)] = jnp.where(pos < npts, lv, _GARB)
            c1 = pltpu.async_copy(fbuf, acc.at[lidx.at[0]], sem_s, add=True)
            c2 = pltpu.async_copy(ones, cnt.at[lidx.at[0]], sem_c, add=True)
            c1.wait()
            c2.wait()
            return carry

        lax.fori_loop(0, nb, batch, 0)
        plsc.subcore_barrier()

        wrows = rows // _NS
        ob = wo_base + tid * wrows
        ab = tid * wrows
        done = 0
        while done < wrows:
            step = min(640, wrows - done)
            pltpu.sync_copy(acc.at[pl.ds(ab + done, step)],
                            wo_sum.at[pl.ds(ob + done, step)])
            done += step
        pltpu.sync_copy(cnt.at[pl.ds(ab, wrows)],
                        wo_cnt.at[pl.ds(ob, wrows)])
        plsc.subcore_barrier()

    for p in range(6):
        if p < 3:
            vrows = _CH if p < 2 else _VOUT - 2 * _CH

            @pl.when(cid == 0)
            def _(p=p, vrows=vrows):
                one_pass(p2v, p * _CH, p * _CH + vrows, vrows,
                         vs_out, vc_out, p * _CH)

        @pl.when(cid == 1)
        def _(p=p):
            one_pass(flat, p * _CH, (p + 1) * _CH, _CH,
                     rs_out, rc_out, p * _CH)


def _sc_scatter(fuse, p2v_s, flat_s, z2d, z1d, zi32):
    mesh = plsc.VectorSubcoreMesh(core_axis_name="c", subcore_axis_name="s")
    f = pl.kernel(
        _scatter_body,
        out_type=[jax.ShapeDtypeStruct((_VOUT, C), jnp.float32),
                  jax.ShapeDtypeStruct((_VOUT,), jnp.float32),
                  jax.ShapeDtypeStruct((_ROUT, C), jnp.float32),
                  jax.ShapeDtypeStruct((_ROUT,), jnp.float32)],
        mesh=mesh,
        scratch_types=[
            pltpu.VMEM((_IB, C), jnp.float32),
            pltpu.VMEM((_PLCAP,), jnp.int32),
            pltpu.VMEM((_PLCAP,), jnp.int32),
            pltpu.VMEM((_PLCAP,), jnp.int32),
            pltpu.VMEM((1, _IB), jnp.int32),
            pltpu.VMEM((_IB,), jnp.float32),
            pltpu.VMEM_SHARED((_ACC, C), jnp.float32),
            pltpu.VMEM_SHARED((_ACC,), jnp.float32),
            pltpu.SemaphoreType.DMA,
            pltpu.SemaphoreType.DMA,
            pltpu.SemaphoreType.DMA,
        ],
        compiler_params=pltpu.CompilerParams(use_tc_tiling_on_sc=False,
                                             needs_layout_passes=False),
    )
    return f(fuse, p2v_s, flat_s, z2d, z1d, zi32)


def _post_v_body(vs_ref, vc_ref, out_ref):
    c = jnp.maximum(vc_ref[...], 1.0)
    out_ref[...] = vs_ref[...] / c[:, None]


def _post_v(vs, vc):
    grid = ((M + 1023) // 1024,)
    return pl.pallas_call(
        _post_v_body,
        grid=grid,
        in_specs=[pl.BlockSpec((1024, C), lambda i: (i, 0)),
                  pl.BlockSpec((1024,), lambda i: (i,))],
        out_specs=pl.BlockSpec((1024, C), lambda i: (i, 0)),
        out_shape=jax.ShapeDtypeStruct((M, C), jnp.float32),
    )(vs, vc)


def _post_r_body(rs_ref, rc_ref, out_ref):
    c = jnp.maximum(rc_ref[...], 1.0)
    out_ref[...] = (rs_ref[...] / c[:, None]).T


def _post_r(rs, rc):
    grid = (HW // 1024,)
    return pl.pallas_call(
        _post_r_body,
        grid=grid,
        in_specs=[pl.BlockSpec((1024, C), lambda i: (i, 0)),
                  pl.BlockSpec((1024,), lambda i: (i,))],
        out_specs=pl.BlockSpec((C, 1024), lambda i: (0, i)),
        out_shape=jax.ShapeDtypeStruct((C, HW), jnp.float32),
    )(rs, rc)


def kernel(r, p_F, v_F, W_r, b_r, W_p, b_p, W_v, b_v, p2v, px, py):
    flat = py * Wd + px
    rT = r.reshape(C, HW).T
    p2v_pad = jnp.pad(p2v, (0, _NPAD - N)).reshape(_NW, _NPW // _IB, _IB)
    flat_pad = jnp.pad(flat, (0, _NPAD - N)).reshape(_NW, _NPW // _IB, _IB)
    z2d = jnp.zeros((_SB, C), jnp.float32)
    z1d = jnp.zeros((1600,), jnp.float32)
    zi32 = jnp.zeros((_PLCAP,), jnp.int32)
    sentinel = jnp.int32(1 << 20)
    p2v_s = jnp.pad(p2v, (0, _NPAD - N), constant_values=sentinel)
    flat_s = jnp.pad(flat, (0, _NPAD - N), constant_values=sentinel)

    v2p_pad, r2p_pad = _sc_gather(v_F, rT, p2v_pad, flat_pad)
    fuse = _fuse_tc(p_F, v2p_pad, r2p_pad, W_r, b_r, W_p, b_p, W_v, b_v)
    vs, vc, rs, rc = _sc_scatter(fuse, p2v_s, flat_s, z2d, z1d, zi32)
    v_new = _post_v(vs, vc)
    r_new = _post_r(rs, rc).reshape(C, H, Wd)
    return (r_new, fuse, v_new)


# speedup vs baseline: 1.7365x; 1.0371x over previous
import jax
import jax.numpy as jnp
from jax import lax
from jax.experimental import pallas as pl
from jax.experimental.pallas import tpu as pltpu
from jax.experimental.pallas import tpu_sc as plsc

N = 100000
M = 50000
C = 64
H = 64
Wd = 2048
HW = H * Wd

_NC = 2
_NS = 16
_NW = _NC * _NS
_NPW = 3200
_NPAD = _NW * _NPW
_IB = 128
_SB = 640
_GB = 320
_GSUP = _NPW // _GB
_GJ = _GB // _IB

_BLK = 2000

_SHARE = 6256
_CH = 22528
_ACC = _CH + 8
_GARB = _CH
_VOUT = 50176
_ROUT = 6 * _CH
_PLCAP = 6272


def _gather_body(vF, rT, p2v, flat, v_out, r_out,
                 idx_v, idx_r, buf_v, buf_r, sem_v0, sem_v1, sem_r0, sem_r1,
                 sem_o):
    wid = lax.axis_index("s") * _NC + lax.axis_index("c")
    base = wid * _NPW
    pltpu.sync_copy(p2v.at[wid], idx_v)
    pltpu.sync_copy(flat.at[wid], idx_r)

    def fire(s, par):
        bo = par * _GB
        sv = sem_v1 if par else sem_v0
        sr = sem_r1 if par else sem_r0
        for j in range(_GB // _IB):
            row = (s * _GB) // _IB + j
            dst = pl.ds(bo + j * _IB, _IB)
            pltpu.async_copy(vF.at[idx_v.at[row]], buf_v.at[dst], sv)
            pltpu.async_copy(rT.at[idx_r.at[row]], buf_r.at[dst], sr)

    def drain(s, par):
        bo = par * _GB
        sv = sem_v1 if par else sem_v0
        sr = sem_r1 if par else sem_r0
        for j in range(_GB // _IB):
            dst = pl.ds(bo + j * _IB, _IB)
            pltpu.make_async_copy(vF.at[idx_v.at[0]], buf_v.at[dst], sv).wait()
            pltpu.make_async_copy(rT.at[idx_r.at[0]], buf_r.at[dst], sr).wait()
        out = pl.ds(base + s * _GB, _GB)
        ov = pltpu.async_copy(buf_v.at[pl.ds(bo, _GB)], v_out.at[out], sem_o)
        orr = pltpu.async_copy(buf_r.at[pl.ds(bo, _GB)], r_out.at[out], sem_o)
        return ov, orr

    fire(0, 0)
    pend = []
    for s in range(_GSUP):
        par = s % 2
        for cp in pend:
            cp.wait()
        pend = []
        if s + 1 < _GSUP:
            fire(s + 1, 1 - par)
        pend = list(drain(s, par))
    for cp in pend:
        cp.wait()


def _sc_gather(v_F, rT, p2v_pad, flat_pad):
    idx_shape = (_NPW // _IB, _IB)
    mesh = plsc.VectorSubcoreMesh(core_axis_name="c", subcore_axis_name="s")
    f = pl.kernel(
        _gather_body,
        out_type=[jax.ShapeDtypeStruct((_NPAD, C), jnp.float32),
                  jax.ShapeDtypeStruct((_NPAD, C), jnp.float32)],
        mesh=mesh,
        scratch_types=[
            pltpu.VMEM(idx_shape, jnp.int32),
            pltpu.VMEM(idx_shape, jnp.int32),
            pltpu.VMEM((2 * _GB, C), jnp.float32),
            pltpu.VMEM((2 * _GB, C), jnp.float32),
            pltpu.SemaphoreType.DMA,
            pltpu.SemaphoreType.DMA,
            pltpu.SemaphoreType.DMA,
            pltpu.SemaphoreType.DMA,
            pltpu.SemaphoreType.DMA,
        ],
        compiler_params=pltpu.CompilerParams(use_tc_tiling_on_sc=False),
    )
    return f(v_F, rT, p2v_pad, flat_pad)


def _tr_body(r_ref, out_ref):
    out_ref[...] = r_ref[...].T


def _tr_tc(r2):
    grid = (HW // 1024,)
    return pl.pallas_call(
        _tr_body,
        grid=grid,
        in_specs=[pl.BlockSpec((C, 1024), lambda i: (0, i))],
        out_specs=pl.BlockSpec((1024, C), lambda i: (i, 0)),
        out_shape=jax.ShapeDtypeStruct((HW, C), jnp.float32),
    )(r2)


def _fuse_body(p_ref, v_ref, r2_ref, wr_ref, br_ref, wp_ref, bp_ref,
               wv_ref, bv_ref, out_ref):
    p = p_ref[...]
    v = v_ref[...]
    r2 = r2_ref[...]
    wr = wr_ref[...]
    wp = wp_ref[...]
    wv = wv_ref[...]
    br = br_ref[...]
    bp = bp_ref[...]
    bv = bv_ref[...]
    cols = []
    for k in range(3):
        acc = (r2 * wr[k][None, :] + p * wp[k][None, :] + v * wv[k][None, :])
        cols.append(jnp.sum(acc, axis=-1) + br[k] + bp[k] + bv[k])
    a0, a1, a2 = cols
    m = jnp.maximum(jnp.maximum(a0, a1), a2)
    e0 = jnp.exp(a0 - m)
    e1 = jnp.exp(a1 - m)
    e2 = jnp.exp(a2 - m)
    inv = 1.0 / (e0 + e1 + e2)
    out_ref[...] = (r2 * (e0 * inv)[:, None] + p * (e1 * inv)[:, None]
                    + v * (e2 * inv)[:, None])


def _fuse_tc(p_F, v2p, r2p, W_r, b_r, W_p, b_p, W_v, b_v):
    grid = (N // _BLK,)
    blk = pl.BlockSpec((_BLK, C), lambda i: (i, 0))
    wspec = pl.BlockSpec((3, C), lambda i: (0, 0))
    bspec = pl.BlockSpec((3,), lambda i: (0,))
    return pl.pallas_call(
        _fuse_body,
        grid=grid,
        in_specs=[blk, blk, blk, wspec, bspec, wspec, bspec, wspec, bspec],
        out_specs=blk,
        out_shape=jax.ShapeDtypeStruct((N, C), jnp.float32),
    )(p_F, v2p, r2p, W_r, b_r, W_p, b_p, W_v, b_v)


def _scatter_body(fuse, p2v, flat, z2d, z1d, zi32,
                  vs_out, vc_out, rs_out, rc_out,
                  fbuf, ibuf, pidl, locl, lidx, ones,
                  acc, cnt, sem_g, sem_g2, sem_s, sem_c):
    cid = lax.axis_index("c")
    tid = lax.axis_index("s")
    base_t = tid * _SHARE

    pltpu.sync_copy(zi32, pidl)
    for k in range(8):
        ones[pl.ds(k * 16, 16)] = jnp.full((16,), 1.0, jnp.float32)
    iota = lax.broadcasted_iota(jnp.int32, (16,), 0)

    def one_pass(src_hbm, lo, hi, rows, wo_sum, wo_cnt, wo_base):
        zrows = rows // _NS
        zb = tid * zrows
        done = 0
        while done < zrows:
            step = min(640, zrows - done)
            pltpu.sync_copy(z2d.at[pl.ds(0, step)],
                            acc.at[pl.ds(zb + done, step)])
            done += step
        pltpu.sync_copy(z1d.at[pl.ds(0, zrows)], cnt.at[pl.ds(zb, zrows)])

        @pl.when(tid == 0)
        def _():
            pltpu.sync_copy(z2d.at[pl.ds(0, 8)], acc.at[pl.ds(_CH, 8)])
            pltpu.sync_copy(z1d.at[pl.ds(0, 8)], cnt.at[pl.ds(_CH, 8)])

        pltpu.sync_copy(src_hbm.at[pl.ds(base_t, _PLCAP)], ibuf)

        def binstep(k, off):
            g = ibuf[pl.ds(k * 16, 16)]
            inb = (g >= lo) & (g < hi) & ((k * 16 + iota) < _SHARE)
            ii = inb.astype(jnp.int32)
            csum = plsc.cumsum(ii)
            tgt = jnp.where(inb, off + csum - ii, 0)
            plsc.store_scatter(pidl, [tgt], base_t + k * 16 + iota, mask=inb)
            plsc.store_scatter(locl, [tgt], g - lo, mask=inb)
            return off + plsc.all_reduce_population_count(inb)

        npts_v = lax.fori_loop(0, _PLCAP // 16, binstep,
                               jnp.zeros((16,), jnp.int32))
        npts = npts_v[0]
        plsc.subcore_barrier()

        nb = (npts + _IB - 1) // _IB

        @pl.when(nb > 0)
        def _():
            pltpu.async_copy(fuse.at[pidl.at[pl.ds(0, _IB)]],
                             fbuf.at[pl.ds(0, _IB)], sem_g)

        def batch(b, carry):
            par = b % 2
            bb = b * _IB

            @pl.when(par == 0)
            def _():
                pltpu.make_async_copy(fuse.at[pidl.at[pl.ds(0, _IB)]],
                                      fbuf.at[pl.ds(0, _IB)], sem_g).wait()

            @pl.when(par == 1)
            def _():
                pltpu.make_async_copy(fuse.at[pidl.at[pl.ds(0, _IB)]],
                                      fbuf.at[pl.ds(_IB, _IB)], sem_g2).wait()

            @pl.when((b + 1 < nb) & (par == 0))
            def _():
                pltpu.async_copy(fuse.at[pidl.at[pl.ds(bb + _IB, _IB)]],
                                 fbuf.at[pl.ds(_IB, _IB)], sem_g2)

            @pl.when((b + 1 < nb) & (par == 1))
            def _():
                pltpu.async_copy(fuse.at[pidl.at[pl.ds(bb + _IB, _IB)]],
                                 fbuf.at[pl.ds(0, _IB)], sem_g)

            for j in range(_IB // 16):
                lv = locl[pl.ds(bb + j * 16, 16)]
                pos = bb + j * 16 + iota
                lidx[0, pl.ds(j * 16, 16)] = jnp.where(pos < npts, lv, _GARB)
            c1 = pltpu.async_copy(fbuf.at[pl.ds(par * _IB, _IB)],
                                  acc.at[lidx.at[0]], sem_s, add=True)
            c2 = pltpu.async_copy(ones, cnt.at[lidx.at[0]], sem_c, add=True)
            c1.wait()
            c2.wait()
            return carry

        lax.fori_loop(0, nb, batch, 0)
        plsc.subcore_barrier()

        wrows = rows // _NS
        ob = wo_base + tid * wrows
        ab = tid * wrows
        done = 0
        while done < wrows:
            step = min(640, wrows - done)
            pltpu.sync_copy(acc.at[pl.ds(ab + done, step)],
                            wo_sum.at[pl.ds(ob + done, step)])
            done += step
        pltpu.sync_copy(cnt.at[pl.ds(ab, wrows)],
                        wo_cnt.at[pl.ds(ob, wrows)])
        plsc.subcore_barrier()

    for p in range(5):
        if p < 3:
            vrows = _CH if p < 2 else _VOUT - 2 * _CH

            @pl.when(cid == 0)
            def _(p=p, vrows=vrows):
                one_pass(p2v, p * _CH, p * _CH + vrows, vrows,
                         vs_out, vc_out, p * _CH)
        elif p == 3:

            @pl.when(cid == 0)
            def _():
                one_pass(flat, 5 * _CH, 6 * _CH, _CH,
                         rs_out, rc_out, 5 * _CH)

        if p < 5:

            @pl.when(cid == 1)
            def _(p=p):
                one_pass(flat, p * _CH, (p + 1) * _CH, _CH,
                         rs_out, rc_out, p * _CH)


def _sc_scatter(fuse, p2v_s, flat_s, z2d, z1d, zi32):
    mesh = plsc.VectorSubcoreMesh(core_axis_name="c", subcore_axis_name="s")
    f = pl.kernel(
        _scatter_body,
        out_type=[jax.ShapeDtypeStruct((_VOUT, C), jnp.float32),
                  jax.ShapeDtypeStruct((_VOUT,), jnp.float32),
                  jax.ShapeDtypeStruct((_ROUT, C), jnp.float32),
                  jax.ShapeDtypeStruct((_ROUT,), jnp.float32)],
        mesh=mesh,
        scratch_types=[
            pltpu.VMEM((2 * _IB, C), jnp.float32),
            pltpu.VMEM((_PLCAP,), jnp.int32),
            pltpu.VMEM((_PLCAP,), jnp.int32),
            pltpu.VMEM((_PLCAP,), jnp.int32),
            pltpu.VMEM((1, _IB), jnp.int32),
            pltpu.VMEM((_IB,), jnp.float32),
            pltpu.VMEM_SHARED((_ACC, C), jnp.float32),
            pltpu.VMEM_SHARED((_ACC,), jnp.float32),
            pltpu.SemaphoreType.DMA,
            pltpu.SemaphoreType.DMA,
            pltpu.SemaphoreType.DMA,
            pltpu.SemaphoreType.DMA,
        ],
        compiler_params=pltpu.CompilerParams(use_tc_tiling_on_sc=False,
                                             needs_layout_passes=False),
    )
    return f(fuse, p2v_s, flat_s, z2d, z1d, zi32)


def _post_v_body(vs_ref, vc_ref, out_ref):
    c = jnp.maximum(vc_ref[...], 1.0)
    out_ref[...] = vs_ref[...] / c[:, None]


def _post_v(vs, vc):
    grid = ((M + 1023) // 1024,)
    return pl.pallas_call(
        _post_v_body,
        grid=grid,
        in_specs=[pl.BlockSpec((1024, C), lambda i: (i, 0)),
                  pl.BlockSpec((1024,), lambda i: (i,))],
        out_specs=pl.BlockSpec((1024, C), lambda i: (i, 0)),
        out_shape=jax.ShapeDtypeStruct((M, C), jnp.float32),
    )(vs, vc)


def _post_r_body(rs_ref, rc_ref, out_ref):
    c = jnp.maximum(rc_ref[...], 1.0)
    out_ref[...] = (rs_ref[...] / c[:, None]).T


def _post_r(rs, rc):
    grid = (HW // 1024,)
    return pl.pallas_call(
        _post_r_body,
        grid=grid,
        in_specs=[pl.BlockSpec((1024, C), lambda i: (i, 0)),
                  pl.BlockSpec((1024,), lambda i: (i,))],
        out_specs=pl.BlockSpec((C, 1024), lambda i: (0, i)),
        out_shape=jax.ShapeDtypeStruct((C, HW), jnp.float32),
    )(rs, rc)


def kernel(r, p_F, v_F, W_r, b_r, W_p, b_p, W_v, b_v, p2v, px, py):
    flat = py * Wd + px
    rT = r.reshape(C, HW).T
    p2v_pad = jnp.pad(p2v, (0, _NPAD - N)).reshape(_NW, _NPW // _IB, _IB)
    flat_pad = jnp.pad(flat, (0, _NPAD - N)).reshape(_NW, _NPW // _IB, _IB)
    z2d = jnp.zeros((_SB, C), jnp.float32)
    z1d = jnp.zeros((1600,), jnp.float32)
    zi32 = jnp.zeros((_PLCAP,), jnp.int32)
    sentinel = jnp.int32(1 << 20)
    p2v_s = jnp.pad(p2v, (0, _NPAD - N), constant_values=sentinel)
    flat_s = jnp.pad(flat, (0, _NPAD - N), constant_values=sentinel)

    v2p_pad, r2p_pad = _sc_gather(v_F, rT, p2v_pad, flat_pad)
    fuse = _fuse_tc(p_F, v2p_pad, r2p_pad, W_r, b_r, W_p, b_p, W_v, b_v)
    vs, vc, rs, rc = _sc_scatter(fuse, p2v_s, flat_s, z2d, z1d, zi32)
    v_new = _post_v(vs, vc)
    r_new = _post_r(rs, rc).reshape(C, H, Wd)
    return (r_new, fuse, v_new)
